# baseline (speedup 1.0000x reference)
import functools

import jax
import jax.numpy as jnp
from jax import lax
from jax.experimental import pallas as pl
from jax.experimental.pallas import tpu as pltpu
from jax.experimental.pallas import tpu_sc as plsc

_B = 256
_W = 4
_NCH = 4
_NCO = 3
_D = 300
_NWORD = 2010
_NCOMPO = 20000
_NC = 2
_NS = 16
_NW = _NC * _NS
_BPW = _B // _NW
_L = 16
_NFULL = _D // _L
_TAIL = _D - _NFULL * _L

_VB = 2048
_NA = -(-_NCOMPO // _VB)
_NB = 8
_NBLK = (_B * _W) // _NB


def _sc_body(chidx_hbm, widx_hbm, char_hbm, word_hbm,
             tgt_out, wctx_out,
             chidx_v, widx_v, chrows_v, wrows_v, tacc_v,
             hsem, wsem):
    wid = lax.axis_index("s") * _NC + lax.axis_index("c")
    nch = _BPW * _NCH
    nw = _BPW * _W

    pltpu.sync_copy(chidx_hbm.at[pl.ds(wid * nch, nch)], chidx_v)
    pltpu.sync_copy(widx_hbm.at[pl.ds(wid * nw, nw)], widx_v)

    def _scalars(ref, n):
        vals = [None] * n
        starts = sorted({*range(0, n - _L + 1, _L), n - _L})
        for s in starts:
            v = ref[pl.ds(s, _L)]
            for l in range(_L):
                if vals[s + l] is None:
                    vals[s + l] = v[l]
        return vals

    hids = _scalars(chidx_v, nch)
    wids = _scalars(widx_v, nw)

    hd = [pltpu.async_copy(char_hbm.at[pl.ds(hids[r], 1)],
                           chrows_v.at[pl.ds(r, 1)], hsem)
          for r in range(nch)]
    wd = [pltpu.async_copy(word_hbm.at[pl.ds(wids[r], 1)],
                           wrows_v.at[pl.ds(r, 1)], wsem)
          for r in range(nw)]

    offs = [k * _L for k in range(_NFULL)] + [_D - _L]

    for d in hd:
        d.wait()

    for b in range(_BPW):
        acc = [jnp.zeros((_L,), jnp.float32) for _ in range(len(offs))]
        for j in range(_NCH):
            r = b * _NCH + j
            m = jnp.where(hids[r] != 1, 1.0, 0.0)
            for k, o in enumerate(offs):
                acc[k] = acc[k] + chrows_v[r, pl.ds(o, _L)] * m
        for k, o in enumerate(offs):
            tacc_v[b, pl.ds(o, _L)] = acc[k]

    pltpu.sync_copy(tacc_v, tgt_out.at[pl.ds(wid * _BPW, _BPW)])

    for d in wd:
        d.wait()
    pltpu.sync_copy(wrows_v, wctx_out.at[pl.ds(wid * nw, nw)])


@functools.lru_cache(maxsize=1)
def _get_sc_gather():
    return functools.partial(
        pl.kernel,
        out_type=(jax.ShapeDtypeStruct((_B, _D), jnp.float32),
                  jax.ShapeDtypeStruct((_B * _W, _D), jnp.float32)),
        mesh=plsc.VectorSubcoreMesh(core_axis_name="c", subcore_axis_name="s"),
        scratch_types=[
            pltpu.VMEM((_BPW * _NCH,), jnp.int32),
            pltpu.VMEM((_BPW * _W,), jnp.int32),
            pltpu.VMEM((_BPW * _NCH, _D), jnp.float32),
            pltpu.VMEM((_BPW * _W, _D), jnp.float32),
            pltpu.VMEM((_BPW, _D), jnp.float32),
            pltpu.SemaphoreType.DMA,
            pltpu.SemaphoreType.DMA,
        ],
    )(_sc_body)


def _tc_body(cw_ref, cm_ref, tgtch_ref, wctx_ref, compot_ref, noise_ref,
             out_ref, tgtct_acc, tgt_acc):
    i = pl.program_id(0)

    @pl.when(i < _NA)
    def _phase_a():
        v0 = i * _VB
        iota_v = lax.broadcasted_iota(jnp.int32, (_VB, _B), 0) + v0
        cm = cm_ref[...]
        o = jnp.zeros((_VB, _B), jnp.bfloat16)
        for j in range(_NCO):
            ids = cm[:, j][None, :]
            hit = (iota_v == ids) & (ids != 1)
            o = o + hit.astype(jnp.bfloat16)
        blk = compot_ref[...]
        blk = jnp.where(jnp.isfinite(blk), blk, 0.0).astype(jnp.bfloat16)
        part = lax.dot_general(blk, o, (((1,), (0,)), ((), ())),
                               preferred_element_type=jnp.float32)

        @pl.when(i == 0)
        def _():
            tgtct_acc[...] = part

        @pl.when(i > 0)
        def _():
            tgtct_acc[...] = tgtct_acc[...] + part

    @pl.when(i == _NA)
    def _start_b():
        tgt = tgtch_ref[...] + tgtct_acc[...].T
        tgt_acc[...] = tgt
        dfull = lax.dot_general(tgt, wctx_ref[...], (((1,), (1,)), ((), ())),
                                preferred_element_type=jnp.float32,
                                precision=lax.Precision.HIGHEST)
        row = lax.broadcasted_iota(jnp.int32, (_B, _B * _W), 0)
        col = lax.broadcasted_iota(jnp.int32, (_B, _B * _W), 1)
        bd = (col // _W) == row
        masked = jnp.where(bd, dfull, 0.0)
        gsel = ((lax.broadcasted_iota(jnp.int32, (_B * _W, _W), 0) % _W) ==
                lax.broadcasted_iota(jnp.int32, (_B * _W, _W), 1)
                ).astype(jnp.float32)
        dots = lax.dot_general(masked, gsel, (((1,), (0,)), ((), ())),
                               preferred_element_type=jnp.float32,
                               precision=lax.Precision.HIGHEST)
        mask = (cw_ref[...] != 1).astype(jnp.float32)
        dots = dots * mask
        sd = 1.0 / (1.0 + jnp.exp(-dots))
        out_ref[...] = jnp.sum(jnp.log(sd)).reshape(1, 1)

    @pl.when(i >= _NA)
    def _phase_b():
        nf = noise_ref[...].astype(jnp.float32)
        s = -lax.dot_general(tgt_acc[...], nf, (((1,), (1,)), ((), ())),
                             preferred_element_type=jnp.float32,
                             precision=lax.Precision.HIGHEST)
        sig = 1.0 / (1.0 + jnp.exp(-s))
        out_ref[...] = out_ref[...] + jnp.sum(jnp.log(sig + 1e-32))

    @pl.when(i == _NA + _NB - 1)
    def _fin():
        out_ref[...] = out_ref[...] * (-1.0 / _B)


_tc_loss = pl.pallas_call(
    _tc_body,
    grid=(_NA + _NB,),
    in_specs=[
        pl.BlockSpec((_B, _W), lambda i: (0, 0)),
        pl.BlockSpec((_B, _NCO), lambda i: (0, 0)),
        pl.BlockSpec((_B, _D), lambda i: (0, 0)),
        pl.BlockSpec((_B * _W, _D), lambda i: (0, 0)),
        pl.BlockSpec((_D, _VB), lambda i: (0, jnp.minimum(i, _NA - 1))),
        pl.BlockSpec((_NBLK, _D), lambda i: (jnp.maximum(i - _NA, 0), 0)),
    ],
    out_specs=pl.BlockSpec((1, 1), lambda i: (0, 0)),
    out_shape=jax.ShapeDtypeStruct((1, 1), jnp.float32),
    scratch_shapes=[
        pltpu.VMEM((_D, _B), jnp.float32),
        pltpu.VMEM((_B, _D), jnp.float32),
    ],
)


def kernel(tgt_chars, tgt_compos, ctx_words, noise, word_emb, char_emb,
           compo_emb):
    chidx = tgt_chars.reshape(-1).astype(jnp.int32)
    widx = ctx_words.reshape(-1).astype(jnp.int32)
    tgt_ch, wctx = _get_sc_gather()(chidx, widx, char_emb, word_emb)
    noise2 = noise.reshape(_B * _W, _D).astype(jnp.int32)
    compot = compo_emb.T
    loss2d = _tc_loss(ctx_words.astype(jnp.int32),
                      tgt_compos.astype(jnp.int32), tgt_ch, wctx, compot,
                      noise2)
    return loss2d[0, 0]


# speedup vs baseline: 1.0941x; 1.0941x over previous
import functools

import jax
import jax.numpy as jnp
from jax import lax
from jax.experimental import pallas as pl
from jax.experimental.pallas import tpu as pltpu
from jax.experimental.pallas import tpu_sc as plsc

_B = 256
_W = 4
_NCH = 4
_NCO = 3
_D = 300
_NWORD = 2010
_NCOMPO = 20000
_NC = 2
_NS = 16
_NW = _NC * _NS
_BPW = _B // _NW
_L = 16
_NFULL = _D // _L
_TAIL = _D - _NFULL * _L

_VB = 2048
_NA = -(-_NCOMPO // _VB)
_NB = 8
_NBLK = (_B * _W) // _NB


def _sc_body(chidx_hbm, widx_hbm, char_hbm, word_hbm,
             tgt_out, wctx_out,
             chidx_v, widx_v, chrows_v, wrows_v, tacc_v,
             hsem, wsem):
    wid = lax.axis_index("s") * _NC + lax.axis_index("c")
    nch = _BPW * _NCH
    nw = _BPW * _W

    pltpu.sync_copy(chidx_hbm.at[pl.ds(wid * nch, nch)], chidx_v)
    pltpu.sync_copy(widx_hbm.at[pl.ds(wid * nw, nw)], widx_v)

    def _scalars(ref, n):
        vals = [None] * n
        starts = sorted({*range(0, n - _L + 1, _L), n - _L})
        for s in starts:
            v = ref[pl.ds(s, _L)]
            for l in range(_L):
                if vals[s + l] is None:
                    vals[s + l] = v[l]
        return vals

    hids = _scalars(chidx_v, nch)
    wids = _scalars(widx_v, nw)

    hd = [pltpu.async_copy(char_hbm.at[pl.ds(hids[r], 1)],
                           chrows_v.at[pl.ds(r, 1)], hsem)
          for r in range(nch)]
    wd = [pltpu.async_copy(word_hbm.at[pl.ds(wids[r], 1)],
                           wrows_v.at[pl.ds(r, 1)], wsem)
          for r in range(nw)]

    offs = [k * _L for k in range(_NFULL)] + [_D - _L]

    for d in hd:
        d.wait()

    for b in range(_BPW):
        acc = [jnp.zeros((_L,), jnp.float32) for _ in range(len(offs))]
        for j in range(_NCH):
            r = b * _NCH + j
            m = jnp.where(hids[r] != 1, 1.0, 0.0)
            for k, o in enumerate(offs):
                acc[k] = acc[k] + chrows_v[r, pl.ds(o, _L)] * m
        for k, o in enumerate(offs):
            tacc_v[b, pl.ds(o, _L)] = acc[k]

    pltpu.sync_copy(tacc_v, tgt_out.at[pl.ds(wid * _BPW, _BPW)])

    for d in wd:
        d.wait()
    pltpu.sync_copy(wrows_v, wctx_out.at[pl.ds(wid * nw, nw)])


@functools.lru_cache(maxsize=1)
def _get_sc_gather():
    return functools.partial(
        pl.kernel,
        out_type=(jax.ShapeDtypeStruct((_B, _D), jnp.float32),
                  jax.ShapeDtypeStruct((_B * _W, _D), jnp.float32)),
        mesh=plsc.VectorSubcoreMesh(core_axis_name="c", subcore_axis_name="s"),
        scratch_types=[
            pltpu.VMEM((_BPW * _NCH,), jnp.int32),
            pltpu.VMEM((_BPW * _W,), jnp.int32),
            pltpu.VMEM((_BPW * _NCH, _D), jnp.float32),
            pltpu.VMEM((_BPW * _W, _D), jnp.float32),
            pltpu.VMEM((_BPW, _D), jnp.float32),
            pltpu.SemaphoreType.DMA,
            pltpu.SemaphoreType.DMA,
        ],
    )(_sc_body)


def _tca_body(cm_ref, compot_ref, out_ref):
    i = pl.program_id(0)
    v0 = i * _VB
    iota16 = lax.broadcasted_iota(jnp.int16, (_VB, _B), 0)
    cm = cm_ref[...]
    cma = jnp.where(cm == 1, -2, cm) - v0
    o = jnp.zeros((_VB, _B), jnp.bfloat16)
    for j in range(_NCO):
        ids16 = cma[:, j].astype(jnp.int16)[None, :]
        o = o + (iota16 == ids16).astype(jnp.bfloat16)
    blk = compot_ref[...]
    blk = jnp.where(jnp.isfinite(blk), blk, 0.0).astype(jnp.bfloat16)
    part = lax.dot_general(blk, o, (((1,), (0,)), ((), ())),
                           preferred_element_type=jnp.float32)

    @pl.when(i == 0)
    def _():
        out_ref[...] = part

    @pl.when(i > 0)
    def _():
        out_ref[...] = out_ref[...] + part


_tc_compo = pl.pallas_call(
    _tca_body,
    grid=(_NA,),
    in_specs=[
        pl.BlockSpec((_B, _NCO), lambda i: (0, 0)),
        pl.BlockSpec((_D, _VB), lambda i: (0, i)),
    ],
    out_specs=pl.BlockSpec((_D, _B), lambda i: (0, 0)),
    out_shape=jax.ShapeDtypeStruct((_D, _B), jnp.float32),
)


def _tcb_body(cw_ref, tgtch_ref, tgtct_ref, wctx_ref, noise_ref,
              out_ref, tgtb_acc):
    i = pl.program_id(0)

    @pl.when(i == 0)
    def _start():
        tgt = tgtch_ref[...] + tgtct_ref[...].T
        tgtb = tgt.astype(jnp.bfloat16)
        tgtb_acc[...] = tgtb
        dfull = lax.dot_general(tgtb, wctx_ref[...].astype(jnp.bfloat16),
                                (((1,), (1,)), ((), ())),
                                preferred_element_type=jnp.float32)
        row = lax.broadcasted_iota(jnp.int32, (_B, _B * _W), 0)
        col = lax.broadcasted_iota(jnp.int32, (_B, _B * _W), 1)
        masked = jnp.where((col // _W) == row, dfull, 0.0)
        gsel = ((lax.broadcasted_iota(jnp.int32, (_B * _W, _W), 0) % _W) ==
                lax.broadcasted_iota(jnp.int32, (_B * _W, _W), 1)
                ).astype(jnp.bfloat16)
        dots = lax.dot_general(masked.astype(jnp.bfloat16), gsel,
                               (((1,), (0,)), ((), ())),
                               preferred_element_type=jnp.float32)
        mask = (cw_ref[...] != 1).astype(jnp.float32)
        dots = dots * mask
        sd = 1.0 / (1.0 + jnp.exp(-dots))
        out_ref[...] = jnp.sum(jnp.log(sd)).reshape(1, 1)

    nf = noise_ref[...].astype(jnp.bfloat16)
    s = -lax.dot_general(tgtb_acc[...], nf, (((1,), (1,)), ((), ())),
                         preferred_element_type=jnp.float32)
    sig = 1.0 / (1.0 + jnp.exp(-s))
    out_ref[...] = out_ref[...] + jnp.sum(jnp.log(sig + 1e-32))

    @pl.when(i == _NB - 1)
    def _fin():
        out_ref[...] = out_ref[...] * (-1.0 / _B)


_tc_loss = pl.pallas_call(
    _tcb_body,
    grid=(_NB,),
    in_specs=[
        pl.BlockSpec((_B, _W), lambda i: (0, 0)),
        pl.BlockSpec((_B, _D), lambda i: (0, 0)),
        pl.BlockSpec((_D, _B), lambda i: (0, 0)),
        pl.BlockSpec((_B * _W, _D), lambda i: (0, 0)),
        pl.BlockSpec((_NBLK, _D), lambda i: (i, 0)),
    ],
    out_specs=pl.BlockSpec((1, 1), lambda i: (0, 0)),
    out_shape=jax.ShapeDtypeStruct((1, 1), jnp.float32),
    scratch_shapes=[
        pltpu.VMEM((_B, _D), jnp.bfloat16),
    ],
)


def kernel(tgt_chars, tgt_compos, ctx_words, noise, word_emb, char_emb,
           compo_emb):
    chidx = tgt_chars.reshape(-1).astype(jnp.int32)
    widx = ctx_words.reshape(-1).astype(jnp.int32)
    tgt_ch, wctx = _get_sc_gather()(chidx, widx, char_emb, word_emb)
    noise2 = noise.reshape(_B * _W, _D).astype(jnp.int32)
    compot = compo_emb.T
    tgt_ct = _tc_compo(tgt_compos.astype(jnp.int32), compot)
    loss2d = _tc_loss(ctx_words.astype(jnp.int32), tgt_ch, tgt_ct, wctx,
                      noise2)
    return loss2d[0, 0]


# speedup vs baseline: 1.2161x; 1.1116x over previous
import functools

import jax
import jax.numpy as jnp
from jax import lax
from jax.experimental import pallas as pl
from jax.experimental.pallas import tpu as pltpu
from jax.experimental.pallas import tpu_sc as plsc

_B = 256
_W = 4
_NCH = 4
_NCO = 3
_D = 300
_NWORD = 2010
_NCOMPO = 20000
_NC = 2
_NS = 16
_NW = _NC * _NS
_BPW = _B // _NW
_L = 16
_NFULL = _D // _L
_TAIL = _D - _NFULL * _L

_VB = 2048
_NA = -(-_NCOMPO // _VB)
_NB = 8
_NBLK = (_B * _W) // _NB


def _sc_body(chidx_hbm, widx_hbm, char_hbm, word_hbm,
             tgt_out, wctx_out,
             chidx_v, widx_v, chrows_v, wrows_v, tacc_v,
             hsem, wsem):
    wid = lax.axis_index("s") * _NC + lax.axis_index("c")
    nch = _BPW * _NCH
    nw = _BPW * _W

    pltpu.sync_copy(chidx_hbm.at[pl.ds(wid * nch, nch)], chidx_v)
    pltpu.sync_copy(widx_hbm.at[pl.ds(wid * nw, nw)], widx_v)

    def _scalars(ref, n):
        vals = [None] * n
        starts = sorted({*range(0, n - _L + 1, _L), n - _L})
        for s in starts:
            v = ref[pl.ds(s, _L)]
            for l in range(_L):
                if vals[s + l] is None:
                    vals[s + l] = v[l]
        return vals

    hids = _scalars(chidx_v, nch)
    wids = _scalars(widx_v, nw)

    hd = [pltpu.async_copy(char_hbm.at[pl.ds(hids[r], 1)],
                           chrows_v.at[pl.ds(r, 1)], hsem)
          for r in range(nch)]
    wd = [pltpu.async_copy(word_hbm.at[pl.ds(wids[r], 1)],
                           wrows_v.at[pl.ds(r, 1)], wsem)
          for r in range(nw)]

    offs = [k * _L for k in range(_NFULL)] + [_D - _L]

    for d in hd:
        d.wait()

    for b in range(_BPW):
        acc = [jnp.zeros((_L,), jnp.float32) for _ in range(len(offs))]
        for j in range(_NCH):
            r = b * _NCH + j
            m = jnp.where(hids[r] != 1, 1.0, 0.0)
            for k, o in enumerate(offs):
                acc[k] = acc[k] + chrows_v[r, pl.ds(o, _L)] * m
        for k, o in enumerate(offs):
            tacc_v[b, pl.ds(o, _L)] = acc[k]

    pltpu.sync_copy(tacc_v, tgt_out.at[pl.ds(wid * _BPW, _BPW)])

    for d in wd:
        d.wait()
    pltpu.sync_copy(wrows_v, wctx_out.at[pl.ds(wid * nw, nw)])


@functools.lru_cache(maxsize=1)
def _get_sc_gather():
    return functools.partial(
        pl.kernel,
        out_type=(jax.ShapeDtypeStruct((_B, _D), jnp.float32),
                  jax.ShapeDtypeStruct((_B * _W, _D), jnp.float32)),
        mesh=plsc.VectorSubcoreMesh(core_axis_name="c", subcore_axis_name="s"),
        scratch_types=[
            pltpu.VMEM((_BPW * _NCH,), jnp.int32),
            pltpu.VMEM((_BPW * _W,), jnp.int32),
            pltpu.VMEM((_BPW * _NCH, _D), jnp.float32),
            pltpu.VMEM((_BPW * _W, _D), jnp.float32),
            pltpu.VMEM((_BPW, _D), jnp.float32),
            pltpu.SemaphoreType.DMA,
            pltpu.SemaphoreType.DMA,
        ],
    )(_sc_body)


def _tca_body(cm_ref, compot_ref, out_ref):
    i = pl.program_id(0)
    v0 = i * _VB
    iota_v = lax.broadcasted_iota(jnp.int32, (_VB, _B), 0)
    cm = cm_ref[...]
    cma = jnp.where(cm == 1, -2, cm) - v0
    o = jnp.zeros((_VB, _B), jnp.float32)
    for j in range(_NCO):
        ids = cma[:, j][None, :]
        o = o + (iota_v == ids).astype(jnp.float32)
    blk = compot_ref[...]
    blk = jnp.where(jnp.isfinite(blk), blk, 0.0).astype(jnp.bfloat16)
    part = lax.dot_general(blk, o.astype(jnp.bfloat16),
                           (((1,), (0,)), ((), ())),
                           preferred_element_type=jnp.float32)

    @pl.when(i == 0)
    def _():
        out_ref[...] = part

    @pl.when(i > 0)
    def _():
        out_ref[...] = out_ref[...] + part


_tc_compo = pl.pallas_call(
    _tca_body,
    grid=(_NA,),
    in_specs=[
        pl.BlockSpec((_B, _NCO), lambda i: (0, 0)),
        pl.BlockSpec((_D, _VB), lambda i: (0, i)),
    ],
    out_specs=pl.BlockSpec((_D, _B), lambda i: (0, 0)),
    out_shape=jax.ShapeDtypeStruct((_D, _B), jnp.float32),
)


def _tcb_body(cw_ref, tgtch_ref, tgtct_ref, wctx_ref, noise_ref,
              out_ref, tgtb_acc):
    i = pl.program_id(0)

    @pl.when(i == 0)
    def _start():
        tgt = tgtch_ref[...] + tgtct_ref[...].T
        tgtb = tgt.astype(jnp.bfloat16)
        tgtb_acc[...] = tgtb
        dfull = lax.dot_general(tgtb, wctx_ref[...].astype(jnp.bfloat16),
                                (((1,), (1,)), ((), ())),
                                preferred_element_type=jnp.float32)
        row = lax.broadcasted_iota(jnp.int32, (_B, _B * _W), 0)
        col = lax.broadcasted_iota(jnp.int32, (_B, _B * _W), 1)
        masked = jnp.where((col // _W) == row, dfull, 0.0)
        gsel = ((lax.broadcasted_iota(jnp.int32, (_B * _W, _W), 0) % _W) ==
                lax.broadcasted_iota(jnp.int32, (_B * _W, _W), 1)
                ).astype(jnp.bfloat16)
        dots = lax.dot_general(masked.astype(jnp.bfloat16), gsel,
                               (((1,), (0,)), ((), ())),
                               preferred_element_type=jnp.float32)
        mask = (cw_ref[...] != 1).astype(jnp.float32)
        dots = dots * mask
        sd = 1.0 / (1.0 + jnp.exp(-dots))
        out_ref[...] = jnp.sum(jnp.log(sd)).reshape(1, 1)

    nf = noise_ref[...].astype(jnp.bfloat16)
    s = -lax.dot_general(tgtb_acc[...], nf, (((1,), (1,)), ((), ())),
                         preferred_element_type=jnp.float32)
    sig = 1.0 / (1.0 + jnp.exp(-s))
    out_ref[...] = out_ref[...] + jnp.sum(jnp.log(sig + 1e-32))

    @pl.when(i == _NB - 1)
    def _fin():
        out_ref[...] = out_ref[...] * (-1.0 / _B)


_tc_loss = pl.pallas_call(
    _tcb_body,
    grid=(_NB,),
    in_specs=[
        pl.BlockSpec((_B, _W), lambda i: (0, 0)),
        pl.BlockSpec((_B, _D), lambda i: (0, 0)),
        pl.BlockSpec((_D, _B), lambda i: (0, 0)),
        pl.BlockSpec((_B * _W, _D), lambda i: (0, 0)),
        pl.BlockSpec((_NBLK, _D), lambda i: (i, 0)),
    ],
    out_specs=pl.BlockSpec((1, 1), lambda i: (0, 0)),
    out_shape=jax.ShapeDtypeStruct((1, 1), jnp.float32),
    scratch_shapes=[
        pltpu.VMEM((_B, _D), jnp.bfloat16),
    ],
)


def kernel(tgt_chars, tgt_compos, ctx_words, noise, word_emb, char_emb,
           compo_emb):
    chidx = tgt_chars.reshape(-1).astype(jnp.int32)
    widx = ctx_words.reshape(-1).astype(jnp.int32)
    tgt_ch, wctx = _get_sc_gather()(chidx, widx, char_emb, word_emb)
    noise2 = noise.reshape(_B * _W, _D).astype(jnp.int32)
    compot = compo_emb.T
    tgt_ct = _tc_compo(tgt_compos.astype(jnp.int32), compot)
    loss2d = _tc_loss(ctx_words.astype(jnp.int32), tgt_ch, tgt_ct, wctx,
                      noise2)
    return loss2d[0, 0]


# speedup vs baseline: 1.3341x; 1.0970x over previous
import functools

import jax
import jax.numpy as jnp
from jax import lax
from jax.experimental import pallas as pl
from jax.experimental.pallas import tpu as pltpu
from jax.experimental.pallas import tpu_sc as plsc

_B = 256
_W = 4
_NCH = 4
_NCO = 3
_D = 300
_NWORD = 2010
_NCOMPO = 20000
_NC = 2
_NS = 16
_NW = _NC * _NS
_BPW = _B // _NW
_L = 16
_NFULL = _D // _L
_TAIL = _D - _NFULL * _L

_VB = 2048
_NA = -(-_NCOMPO // _VB)
_NB = 4
_NBLK = (_B * _W) // _NB


def _sc_body(widx_hbm, word_hbm, wctx_out, widx_v, wrows_v, wsem):
    wid = lax.axis_index("s") * _NC + lax.axis_index("c")
    nw = _BPW * _W

    pltpu.sync_copy(widx_hbm.at[pl.ds(wid * nw, nw)], widx_v)

    def _scalars(ref, n):
        vals = [None] * n
        starts = sorted({*range(0, n - _L + 1, _L), n - _L})
        for s in starts:
            v = ref[pl.ds(s, _L)]
            for l in range(_L):
                if vals[s + l] is None:
                    vals[s + l] = v[l]
        return vals

    wids = _scalars(widx_v, nw)

    wd = [pltpu.async_copy(word_hbm.at[pl.ds(wids[r], 1)],
                           wrows_v.at[pl.ds(r, 1)], wsem)
          for r in range(nw)]
    for d in wd:
        d.wait()
    pltpu.sync_copy(wrows_v, wctx_out.at[pl.ds(wid * nw, nw)])


@functools.lru_cache(maxsize=1)
def _get_sc_gather():
    return functools.partial(
        pl.kernel,
        out_type=jax.ShapeDtypeStruct((_B * _W, _D), jnp.float32),
        mesh=plsc.VectorSubcoreMesh(core_axis_name="c", subcore_axis_name="s"),
        scratch_types=[
            pltpu.VMEM((_BPW * _W,), jnp.int32),
            pltpu.VMEM((_BPW * _W, _D), jnp.float32),
            pltpu.SemaphoreType.DMA,
        ],
    )(_sc_body)


_NAC = -(-5000 // _VB)


def _onehot_accum(ids_ref, tbl_ref, nids, step, out_ref, first):
    v0 = step * _VB
    iota_v = lax.broadcasted_iota(jnp.int32, (_VB, _B), 0)
    cm = ids_ref[...]
    cma = jnp.where(cm == 1, -2, cm) - v0
    o = jnp.zeros((_VB, _B), jnp.float32)
    for j in range(nids):
        ids = cma[:, j][None, :]
        o = o + (iota_v == ids).astype(jnp.float32)
    blk = tbl_ref[...]
    blk = jnp.where(jnp.isfinite(blk), blk, 0.0).astype(jnp.bfloat16)
    part = lax.dot_general(blk, o.astype(jnp.bfloat16),
                           (((1,), (0,)), ((), ())),
                           preferred_element_type=jnp.float32)

    @pl.when(first)
    def _():
        out_ref[...] = part

    @pl.when(jnp.logical_not(first))
    def _():
        out_ref[...] = out_ref[...] + part


def _tca_body(cm_ref, ch_ref, compot_ref, chart_ref, out_ref):
    i = pl.program_id(0)

    @pl.when(i < _NA)
    def _compo():
        _onehot_accum(cm_ref, compot_ref, _NCO, i, out_ref, i == 0)

    @pl.when(i >= _NA)
    def _char():
        _onehot_accum(ch_ref, chart_ref, _NCH, i - _NA, out_ref, i == 0)


_tc_compo = pl.pallas_call(
    _tca_body,
    grid=(_NA + _NAC,),
    in_specs=[
        pl.BlockSpec((_B, _NCO), lambda i: (0, 0)),
        pl.BlockSpec((_B, _NCH), lambda i: (0, 0)),
        pl.BlockSpec((_D, _VB), lambda i: (0, jnp.minimum(i, _NA - 1))),
        pl.BlockSpec((_D, _VB),
                     lambda i: (0, jnp.clip(i - _NA, 0, _NAC - 1))),
    ],
    out_specs=pl.BlockSpec((_D, _B), lambda i: (0, 0)),
    out_shape=jax.ShapeDtypeStruct((_D, _B), jnp.float32),
)


def _tcb_body(cw_ref, tgtct_ref, wctx_ref, noise_ref, out_ref, tgtb_acc):
    i = pl.program_id(0)

    @pl.when(i == 0)
    def _start():
        tgtb = tgtct_ref[...].T.astype(jnp.bfloat16)
        tgtb_acc[...] = tgtb
        dfull = lax.dot_general(tgtb, wctx_ref[...].astype(jnp.bfloat16),
                                (((1,), (1,)), ((), ())),
                                preferred_element_type=jnp.float32)
        row = lax.broadcasted_iota(jnp.int32, (_B, _B * _W), 0)
        col = lax.broadcasted_iota(jnp.int32, (_B, _B * _W), 1)
        masked = jnp.where((col // _W) == row, dfull, 0.0)
        gsel = ((lax.broadcasted_iota(jnp.int32, (_B * _W, _W), 0) % _W) ==
                lax.broadcasted_iota(jnp.int32, (_B * _W, _W), 1)
                ).astype(jnp.bfloat16)
        dots = lax.dot_general(masked.astype(jnp.bfloat16), gsel,
                               (((1,), (0,)), ((), ())),
                               preferred_element_type=jnp.float32)
        mask = (cw_ref[...] != 1).astype(jnp.float32)
        dots = dots * mask
        sd = 1.0 / (1.0 + jnp.exp(-dots))
        out_ref[...] = jnp.sum(jnp.log(sd)).reshape(1, 1)

    nf = noise_ref[...].astype(jnp.bfloat16)
    s = -lax.dot_general(tgtb_acc[...], nf, (((1,), (1,)), ((), ())),
                         preferred_element_type=jnp.float32)
    sig = 1.0 / (1.0 + jnp.exp(-s))
    out_ref[...] = out_ref[...] + jnp.sum(jnp.log(sig + 1e-32))

    @pl.when(i == _NB - 1)
    def _fin():
        out_ref[...] = out_ref[...] * (-1.0 / _B)


_tc_loss = pl.pallas_call(
    _tcb_body,
    grid=(_NB,),
    in_specs=[
        pl.BlockSpec((_B, _W), lambda i: (0, 0)),
        pl.BlockSpec((_D, _B), lambda i: (0, 0)),
        pl.BlockSpec((_B * _W, _D), lambda i: (0, 0)),
        pl.BlockSpec((_NBLK, _D), lambda i: (i, 0)),
    ],
    out_specs=pl.BlockSpec((1, 1), lambda i: (0, 0)),
    out_shape=jax.ShapeDtypeStruct((1, 1), jnp.float32),
    scratch_shapes=[
        pltpu.VMEM((_B, _D), jnp.bfloat16),
    ],
)


def kernel(tgt_chars, tgt_compos, ctx_words, noise, word_emb, char_emb,
           compo_emb):
    widx = ctx_words.reshape(-1).astype(jnp.int32)
    wctx = _get_sc_gather()(widx, word_emb)
    noise2 = noise.reshape(_B * _W, _D).astype(jnp.int32)
    tgt_t = _tc_compo(tgt_compos.astype(jnp.int32),
                      tgt_chars.astype(jnp.int32), compo_emb.T, char_emb.T)
    loss2d = _tc_loss(ctx_words.astype(jnp.int32), tgt_t, wctx, noise2)
    return loss2d[0, 0]
